# baseline (speedup 1.0000x reference)
import functools

import jax
import jax.numpy as jnp
from jax import lax
from jax.experimental import pallas as pl
from jax.experimental.pallas import tpu as pltpu
from jax.experimental.pallas import tpu_sc as plsc

NC = 2
NS = 16
NW = NC * NS


def _sc_mesh():
  return plsc.VectorSubcoreMesh(
      core_axis_name="c", subcore_axis_name="s", num_cores=NC, num_subcores=NS)


_SC_PARAMS = pltpu.CompilerParams(use_tc_tiling_on_sc=False)


def _make_gather(n_rows, d, chunk):
  per_w = n_rows // NW
  assert per_w % chunk == 0 and (chunk * 4) % 8 == 0
  n_ch = per_w // chunk

  @functools.partial(
      pl.kernel,
      mesh=_sc_mesh(),
      compiler_params=_SC_PARAMS,
      out_type=jax.ShapeDtypeStruct((n_rows, d), jnp.float32),
      scratch_types=[
          pltpu.VMEM((chunk,), jnp.int32),
          pltpu.VMEM((chunk, d), jnp.float32),
          pltpu.SemaphoreType.DMA,
      ],
  )
  def gather_kernel(table_hbm, idx_hbm, out_hbm, idx_v, rows_v, sem):
    wid = lax.axis_index("s") * NC + lax.axis_index("c")
    base = wid * per_w
    for j in range(n_ch):
      off = base + j * chunk
      pltpu.sync_copy(idx_hbm.at[pl.ds(off, chunk)], idx_v)
      pltpu.async_copy(table_hbm.at[idx_v], rows_v, sem).wait()
      pltpu.sync_copy(rows_v, out_hbm.at[pl.ds(off, chunk)])

  return gather_kernel


def _make_scatter_add(n_nodes, n_rows, d, chunk):
  per_w = n_rows // NW
  assert per_w % chunk == 0
  n_ch = per_w // chunk

  @functools.partial(
      pl.kernel,
      mesh=_sc_mesh(),
      compiler_params=_SC_PARAMS,
      out_type=jax.ShapeDtypeStruct((NC, n_nodes, d), jnp.float32),
      scratch_types=[
          pltpu.VMEM((chunk,), jnp.int32),
          pltpu.VMEM((chunk, d), jnp.float32),
          pltpu.VMEM_SHARED((n_nodes, d), jnp.float32),
      ],
  )
  def scatter_kernel(rows_hbm, idx_hbm, zeros_hbm, out_hbm, idx_v, rows_v,
                     acc_sh):
    cid = lax.axis_index("c")
    sid = lax.axis_index("s")

    @pl.when(sid == 0)
    def _init():
      pltpu.sync_copy(zeros_hbm, acc_sh)

    plsc.subcore_barrier()
    wid = sid * NC + cid
    base = wid * per_w
    for j in range(n_ch):
      off = base + j * chunk
      pltpu.sync_copy(idx_hbm.at[pl.ds(off, chunk)], idx_v)
      pltpu.sync_copy(rows_hbm.at[pl.ds(off, chunk)], rows_v)
      pltpu.sync_copy(rows_v, acc_sh.at[idx_v], add=True)
    plsc.subcore_barrier()

    @pl.when(sid == 0)
    def _flush():
      pltpu.sync_copy(acc_sh, out_hbm.at[cid])

  return scatter_kernel


def _make_degree(n_nodes, n_rows, chunk):
  per_w = n_rows // NW
  n_ch = per_w // chunk

  @functools.partial(
      pl.kernel,
      mesh=_sc_mesh(),
      compiler_params=_SC_PARAMS,
      out_type=jax.ShapeDtypeStruct((NC, n_nodes, 16), jnp.float32),
      scratch_types=[
          pltpu.VMEM((chunk,), jnp.int32),
          pltpu.VMEM((chunk, 16), jnp.float32),
          pltpu.VMEM_SHARED((n_nodes, 16), jnp.float32),
      ],
  )
  def degree_kernel(idx_hbm, ones_hbm, zeros_hbm, out_hbm, idx_v, ones_v,
                    acc_sh):
    cid = lax.axis_index("c")
    sid = lax.axis_index("s")

    @pl.when(sid == 0)
    def _init():
      pltpu.sync_copy(zeros_hbm, acc_sh)

    pltpu.sync_copy(ones_hbm, ones_v)
    plsc.subcore_barrier()
    wid = sid * NC + cid
    base = wid * per_w
    for j in range(n_ch):
      off = base + j * chunk
      pltpu.sync_copy(idx_hbm.at[pl.ds(off, chunk)], idx_v)
      pltpu.sync_copy(ones_v, acc_sh.at[idx_v], add=True)
    plsc.subcore_barrier()

    @pl.when(sid == 0)
    def _flush():
      pltpu.sync_copy(acc_sh, out_hbm.at[cid])

  return degree_kernel


def _embed(x, W, b):
  n, _ = x.shape
  d = W.shape[1]

  def body(x_ref, w_ref, b_ref, o_ref):
    o_ref[...] = (
        jnp.dot(x_ref[...], w_ref[...], preferred_element_type=jnp.float32)
        + b_ref[...])

  return pl.pallas_call(
      body, out_shape=jax.ShapeDtypeStruct((n, d), jnp.float32))(
          x, W, b.reshape(1, d))


def _edge_messages(ea, xj, K1, Kb1, K2, Kb2, K3, Kb3, R, S, blk):
  e, de = ea.shape
  d = xj.shape[1]
  kw = K1.shape[1]
  dd = K3.shape[1]
  assert e % blk == 0

  def body(ea_ref, xj_ref, k1, kb1, k2, kb2, k3, kb3, r_ref, s_ref, o_ref):
    a = jnp.dot(ea_ref[...], k1[...], preferred_element_type=jnp.float32)
    a = jnp.maximum(a + kb1[...], 0.0)
    a = jnp.dot(a, k2[...], preferred_element_type=jnp.float32)
    a = jnp.maximum(a + kb2[...], 0.0)
    w = jnp.dot(
        a.astype(jnp.bfloat16), k3[...],
        preferred_element_type=jnp.float32) + kb3[...]
    xrep = jnp.dot(
        xj_ref[...].astype(jnp.bfloat16), r_ref[...],
        preferred_element_type=jnp.float32)
    o_ref[...] = jnp.dot(
        (w * xrep).astype(jnp.bfloat16), s_ref[...],
        preferred_element_type=jnp.float32)

  full = lambda shape: pl.BlockSpec(shape, lambda i: (0, 0))
  return pl.pallas_call(
      body,
      grid=(e // blk,),
      in_specs=[
          pl.BlockSpec((blk, de), lambda i: (i, 0)),
          pl.BlockSpec((blk, d), lambda i: (i, 0)),
          full((de, kw)),
          full((1, kw)),
          full((kw, kw)),
          full((1, kw)),
          full((kw, dd)),
          full((1, dd)),
          full((d, dd)),
          full((dd, d)),
      ],
      out_specs=pl.BlockSpec((blk, d), lambda i: (i, 0)),
      out_shape=jax.ShapeDtypeStruct((e, d), jnp.float32),
  )(ea, xj, K1, Kb1.reshape(1, kw), K2, Kb2.reshape(1, kw),
    K3.astype(jnp.bfloat16), Kb3.reshape(1, dd), R.astype(jnp.bfloat16),
    S.astype(jnp.bfloat16))


def _update(agg2, deg2, h, Wr, b):
  n, d = h.shape

  def body(a_ref, d_ref, h_ref, w_ref, b_ref, o_ref):
    agg = a_ref[0] + a_ref[1]
    degs = d_ref[0, :, 0:1] + d_ref[1, :, 0:1]
    rdeg = 1.0 / jnp.maximum(degs, 1.0)
    o_ref[...] = jnp.maximum(
        agg * rdeg
        + jnp.dot(h_ref[...], w_ref[...], preferred_element_type=jnp.float32)
        + b_ref[...], 0.0)

  return pl.pallas_call(
      body, out_shape=jax.ShapeDtypeStruct((n, d), jnp.float32))(
          agg2, deg2, h, Wr, b.reshape(1, d))


def _project(h, W, b):
  n, d = h.shape
  dout = W.shape[1]

  def body(h_ref, w_ref, b_ref, o_ref):
    o_ref[...] = (
        jnp.dot(h_ref[...], w_ref[...], preferred_element_type=jnp.float32)
        + b_ref[...])

  return pl.pallas_call(
      body, out_shape=jax.ShapeDtypeStruct((n, dout), jnp.float32))(
          h, W, b.reshape(1, dout))


def kernel(x, edge_index, edge_attr, W_emb, b_emb, K1, Kb1, K2, Kb2, K3, Kb3,
           W_root, bias, W_inv, b_inv):
  n, _ = x.shape
  e, _ = edge_attr.shape
  d = W_emb.shape[1]

  src = edge_index[0]
  dst = edge_index[1]

  chunk = 1000
  gather = _make_gather(e, d, chunk)
  scatter = _make_scatter_add(n, e, d, chunk)
  degree = _make_degree(n, e, chunk)

  zeros_d = jnp.zeros((n, d), jnp.float32)
  zeros_16 = jnp.zeros((n, 16), jnp.float32)
  ones_16 = jnp.ones((chunk, 16), jnp.float32)

  eye = jnp.eye(d, dtype=jnp.float32)
  R = jnp.kron(eye, jnp.ones((1, d), jnp.float32))
  S = jnp.kron(jnp.ones((d, 1), jnp.float32), eye)

  h = _embed(x, W_emb, b_emb)
  deg2 = degree(dst, ones_16, zeros_16)

  for _ in range(2):
    xj = gather(h, src)
    msg = _edge_messages(edge_attr, xj, K1, Kb1, K2, Kb2, K3, Kb3, R, S, 2000)
    agg2 = scatter(msg, dst, zeros_d)
    h = _update(agg2, deg2, h, W_root, bias)

  return _project(h, W_inv, b_inv)


# speedup vs baseline: 1.5725x; 1.5725x over previous
import functools

import jax
import jax.numpy as jnp
from jax import lax
from jax.experimental import pallas as pl
from jax.experimental.pallas import tpu as pltpu
from jax.experimental.pallas import tpu_sc as plsc

NC = 2
NS = 16
NW = NC * NS


def _sc_mesh():
  return plsc.VectorSubcoreMesh(
      core_axis_name="c", subcore_axis_name="s", num_cores=NC, num_subcores=NS)


_SC_PARAMS = pltpu.CompilerParams(use_tc_tiling_on_sc=False)


def _make_gather(n_rows, d, chunk):
  per_w = n_rows // NW
  assert per_w % chunk == 0 and (chunk * 4) % 8 == 0
  n_ch = per_w // chunk

  @functools.partial(
      pl.kernel,
      mesh=_sc_mesh(),
      compiler_params=_SC_PARAMS,
      out_type=jax.ShapeDtypeStruct((n_rows, d), jnp.float32),
      scratch_types=[
          pltpu.VMEM((chunk,), jnp.int32),
          pltpu.VMEM((chunk, d), jnp.float32),
          pltpu.SemaphoreType.DMA,
      ],
  )
  def gather_kernel(table_hbm, idx_hbm, out_hbm, idx_v, rows_v, sem):
    wid = lax.axis_index("s") * NC + lax.axis_index("c")
    base = wid * per_w
    for j in range(n_ch):
      off = base + j * chunk
      pltpu.sync_copy(idx_hbm.at[pl.ds(off, chunk)], idx_v)
      pltpu.async_copy(table_hbm.at[idx_v], rows_v, sem).wait()
      pltpu.sync_copy(rows_v, out_hbm.at[pl.ds(off, chunk)])

  return gather_kernel


def _make_scatter_add(n_nodes, n_rows, d, chunk):
  per_w = n_rows // NW
  assert per_w % chunk == 0
  n_ch = per_w // chunk

  @functools.partial(
      pl.kernel,
      mesh=_sc_mesh(),
      compiler_params=_SC_PARAMS,
      out_type=jax.ShapeDtypeStruct((NC, n_nodes, d), jnp.float32),
      scratch_types=[
          pltpu.VMEM((chunk,), jnp.int32),
          pltpu.VMEM((chunk, d), jnp.float32),
          pltpu.VMEM_SHARED((n_nodes, d), jnp.float32),
      ],
  )
  def scatter_kernel(rows_hbm, idx_hbm, zeros_hbm, out_hbm, idx_v, rows_v,
                     acc_sh):
    cid = lax.axis_index("c")
    sid = lax.axis_index("s")

    @pl.when(sid == 0)
    def _init():
      pltpu.sync_copy(zeros_hbm, acc_sh)

    plsc.subcore_barrier()
    wid = sid * NC + cid
    base = wid * per_w
    for j in range(n_ch):
      off = base + j * chunk
      pltpu.sync_copy(idx_hbm.at[pl.ds(off, chunk)], idx_v)
      pltpu.sync_copy(rows_hbm.at[pl.ds(off, chunk)], rows_v)
      pltpu.sync_copy(rows_v, acc_sh.at[idx_v], add=True)
    plsc.subcore_barrier()

    @pl.when(sid == 0)
    def _flush():
      pltpu.sync_copy(acc_sh, out_hbm.at[cid])

  return scatter_kernel


def _make_degree(n_nodes, n_rows, chunk):
  per_w = n_rows // NW
  n_ch = per_w // chunk

  @functools.partial(
      pl.kernel,
      mesh=_sc_mesh(),
      compiler_params=_SC_PARAMS,
      out_type=jax.ShapeDtypeStruct((NC, n_nodes, 16), jnp.float32),
      scratch_types=[
          pltpu.VMEM((chunk,), jnp.int32),
          pltpu.VMEM((chunk, 16), jnp.float32),
          pltpu.VMEM_SHARED((n_nodes, 16), jnp.float32),
      ],
  )
  def degree_kernel(idx_hbm, ones_hbm, zeros_hbm, out_hbm, idx_v, ones_v,
                    acc_sh):
    cid = lax.axis_index("c")
    sid = lax.axis_index("s")

    @pl.when(sid == 0)
    def _init():
      pltpu.sync_copy(zeros_hbm, acc_sh)

    pltpu.sync_copy(ones_hbm, ones_v)
    plsc.subcore_barrier()
    wid = sid * NC + cid
    base = wid * per_w
    for j in range(n_ch):
      off = base + j * chunk
      pltpu.sync_copy(idx_hbm.at[pl.ds(off, chunk)], idx_v)
      pltpu.sync_copy(ones_v, acc_sh.at[idx_v], add=True)
    plsc.subcore_barrier()

    @pl.when(sid == 0)
    def _flush():
      pltpu.sync_copy(acc_sh, out_hbm.at[cid])

  return degree_kernel


def _embed(x, W, b):
  n, _ = x.shape
  d = W.shape[1]

  def body(x_ref, w_ref, b_ref, o_ref):
    o_ref[...] = (
        jnp.dot(x_ref[...], w_ref[...], preferred_element_type=jnp.float32)
        + b_ref[...])

  return pl.pallas_call(
      body, out_shape=jax.ShapeDtypeStruct((n, d), jnp.float32))(
          x, W, b.reshape(1, d))


def _edge_messages(eaT, xj, K1, Kb1, K2, Kb2, K3, Kb3, blk):
  de, e = eaT.shape
  d = xj.shape[1]
  kw = K1.shape[1]
  dd = K3.shape[1]
  assert e % blk == 0

  k1t = K1.T
  k2t = K2.T
  k3t = K3.T.astype(jnp.bfloat16)
  kb3t = Kb3.reshape(d, d).T
  kb1c = Kb1.reshape(kw, 1)
  kb2c = Kb2.reshape(kw, 1)

  def body(eaT_ref, xj_ref, k1_ref, kb1_ref, k2_ref, kb2_ref, k3_ref,
           kb3_ref, o_ref):
    a = jnp.dot(k1_ref[...], eaT_ref[...], preferred_element_type=jnp.float32)
    a = jnp.maximum(a + kb1_ref[...], 0.0)
    a = jnp.dot(k2_ref[...], a, preferred_element_type=jnp.float32)
    a = jnp.maximum(a + kb2_ref[...], 0.0)
    wT = jnp.dot(
        k3_ref[...], a.astype(jnp.bfloat16),
        preferred_element_type=jnp.float32).astype(jnp.bfloat16)
    xjT = jnp.transpose(xj_ref[...])
    bias_t = jnp.dot(kb3_ref[...], xjT, preferred_element_type=jnp.float32)
    ch = 640
    for c in range(blk // ch):
      lo = c * ch
      acc = bias_t[:, lo:lo + ch]
      for i in range(d):
        acc = acc + (xjT[i:i + 1, lo:lo + ch] *
                     wT[i * d:(i + 1) * d, lo:lo + ch].astype(jnp.float32))
      o_ref[lo:lo + ch, :] = jnp.transpose(acc)

  full = lambda shape: pl.BlockSpec(shape, lambda i: (0, 0))
  return pl.pallas_call(
      body,
      grid=(e // blk,),
      in_specs=[
          pl.BlockSpec((de, blk), lambda i: (0, i)),
          pl.BlockSpec((blk, d), lambda i: (i, 0)),
          full((kw, de)),
          full((kw, 1)),
          full((kw, kw)),
          full((kw, 1)),
          full((dd, kw)),
          full((d, d)),
      ],
      out_specs=pl.BlockSpec((blk, d), lambda i: (i, 0)),
      out_shape=jax.ShapeDtypeStruct((e, d), jnp.float32),
  )(eaT, xj, k1t, kb1c, k2t, kb2c, k3t, kb3t)


def _update(agg2, deg2, h, Wr, b):
  n, d = h.shape

  def body(a_ref, d_ref, h_ref, w_ref, b_ref, o_ref):
    agg = a_ref[0] + a_ref[1]
    degs = d_ref[0, :, 0:1] + d_ref[1, :, 0:1]
    rdeg = 1.0 / jnp.maximum(degs, 1.0)
    o_ref[...] = jnp.maximum(
        agg * rdeg
        + jnp.dot(h_ref[...], w_ref[...], preferred_element_type=jnp.float32)
        + b_ref[...], 0.0)

  return pl.pallas_call(
      body, out_shape=jax.ShapeDtypeStruct((n, d), jnp.float32))(
          agg2, deg2, h, Wr, b.reshape(1, d))


def _project(h, W, b):
  n, d = h.shape
  dout = W.shape[1]

  def body(h_ref, w_ref, b_ref, o_ref):
    o_ref[...] = (
        jnp.dot(h_ref[...], w_ref[...], preferred_element_type=jnp.float32)
        + b_ref[...])

  return pl.pallas_call(
      body, out_shape=jax.ShapeDtypeStruct((n, dout), jnp.float32))(
          h, W, b.reshape(1, dout))


def kernel(x, edge_index, edge_attr, W_emb, b_emb, K1, Kb1, K2, Kb2, K3, Kb3,
           W_root, bias, W_inv, b_inv):
  n, _ = x.shape
  e, _ = edge_attr.shape
  d = W_emb.shape[1]

  src = edge_index[0]
  dst = edge_index[1]

  chunk = 1000
  gather = _make_gather(e, d, chunk)
  scatter = _make_scatter_add(n, e, d, chunk)
  degree = _make_degree(n, e, chunk)

  zeros_d = jnp.zeros((n, d), jnp.float32)
  zeros_16 = jnp.zeros((n, 16), jnp.float32)
  ones_16 = jnp.ones((chunk, 16), jnp.float32)

  eaT = edge_attr.T

  h = _embed(x, W_emb, b_emb)
  deg2 = degree(dst, ones_16, zeros_16)

  for _ in range(2):
    xj = gather(h, src)
    msg = _edge_messages(eaT, xj, K1, Kb1, K2, Kb2, K3, Kb3, 3200)
    agg2 = scatter(msg, dst, zeros_d)
    h = _update(agg2, deg2, h, W_root, bias)

  return _project(h, W_inv, b_inv)
